# Initial kernel scaffold; baseline (speedup 1.0000x reference)
import jax
import jax.numpy as jnp
from jax.experimental import pallas as pl


def kernel(x, Wr, br, We, be):
    raise NotImplementedError("write your pallas kernel here")


# speedup vs baseline: 2.4034x; 2.4034x over previous
import functools

import jax
import jax.numpy as jnp
from jax import lax
from jax.experimental import pallas as pl
from jax.experimental.pallas import tpu as pltpu


def _moe_body(x_ref, wr_ref, br_ref, we_ref, be_ref, y_ref, aux_ref,
              w_scr, psum_scr, fsum_scr, *, n_i, n_e, bm, E, K, T):
    i = pl.program_id(0)
    e = pl.program_id(1)

    @pl.when(e == 0)
    def _router():
        xb = x_ref[...]
        logits = jnp.dot(xb, wr_ref[...], preferred_element_type=jnp.float32)
        logits = logits + br_ref[...]
        m = jnp.max(logits, axis=1, keepdims=True)
        p = jnp.exp(logits - m)
        z = jnp.sum(p, axis=1, keepdims=True)
        probs = p / z
        ei = lax.broadcasted_iota(jnp.int32, (bm, E), 1)
        p1 = jnp.max(probs, axis=1, keepdims=True)
        idx1 = jnp.min(jnp.where(probs == p1, ei, E), axis=1, keepdims=True)
        m1 = ei == idx1
        probs2 = jnp.where(m1, -1.0, probs)
        p2 = jnp.max(probs2, axis=1, keepdims=True)
        idx2 = jnp.min(jnp.where(probs2 == p2, ei, E), axis=1, keepdims=True)
        m2 = ei == idx2
        denom = p1 + p2 + 1e-9
        w = (p1 / denom) * m1.astype(jnp.float32) + (p2 / denom) * m2.astype(jnp.float32)
        w_scr[...] = w

        psum = jnp.sum(probs, axis=0, keepdims=True)
        fsum = jnp.sum(m1.astype(jnp.float32) + m2.astype(jnp.float32),
                       axis=0, keepdims=True)

        @pl.when(i == 0)
        def _init():
            psum_scr[...] = psum
            fsum_scr[...] = fsum

        @pl.when(i > 0)
        def _acc():
            psum_scr[...] += psum
            fsum_scr[...] += fsum

        @pl.when(i == n_i - 1)
        def _aux():
            scale = float(E) / (float(T) * float(T) * float(K))
            aux_ref[...] = (scale * jnp.sum(psum_scr[...] * fsum_scr[...])
                            ).reshape(1, 1)

    lane = lax.broadcasted_iota(jnp.int32, (bm, E), 1)
    w_col = jnp.sum(jnp.where(lane == e, w_scr[...], 0.0),
                    axis=1, keepdims=True)
    xw = x_ref[...] * w_col
    contrib = jnp.dot(xw, we_ref[0], preferred_element_type=jnp.float32)
    contrib = contrib + w_col * be_ref[0]

    @pl.when(e == 0)
    def _y0():
        y_ref[...] = contrib

    @pl.when(e > 0)
    def _yacc():
        y_ref[...] += contrib


def kernel(x, Wr, br, We, be):
    B, S, D = x.shape
    T = B * S
    E = Wr.shape[1]
    D_OUT = We.shape[2]
    K = 2
    BM = 1024
    n_i = T // BM

    x_flat = x.reshape(T, D)
    br2 = br.reshape(1, E)
    be3 = be.reshape(E, 1, D_OUT)

    grid = (n_i, E)
    y, aux = pl.pallas_call(
        functools.partial(_moe_body, n_i=n_i, n_e=E, bm=BM, E=E, K=K, T=T),
        grid=grid,
        in_specs=[
            pl.BlockSpec((BM, D), lambda i, e: (i, 0)),
            pl.BlockSpec((D, E), lambda i, e: (0, 0)),
            pl.BlockSpec((1, E), lambda i, e: (0, 0)),
            pl.BlockSpec((1, D, D_OUT), lambda i, e: (e, 0, 0)),
            pl.BlockSpec((1, 1, D_OUT), lambda i, e: (e, 0, 0)),
        ],
        out_specs=[
            pl.BlockSpec((BM, D_OUT), lambda i, e: (i, 0)),
            pl.BlockSpec((1, 1), lambda i, e: (0, 0)),
        ],
        out_shape=[
            jax.ShapeDtypeStruct((T, D_OUT), jnp.float32),
            jax.ShapeDtypeStruct((1, 1), jnp.float32),
        ],
        scratch_shapes=[
            pltpu.VMEM((BM, E), jnp.float32),
            pltpu.VMEM((1, E), jnp.float32),
            pltpu.VMEM((1, E), jnp.float32),
        ],
        compiler_params=pltpu.CompilerParams(
            dimension_semantics=("arbitrary", "arbitrary"),
        ),
    )(x_flat, Wr, br2, We, be3)

    return y.reshape(B, S, D_OUT), aux.reshape(())
